# baseline (speedup 1.0000x reference)
import functools

import jax
import jax.numpy as jnp
from jax import lax
from jax.experimental import pallas as pl
from jax.experimental.pallas import tpu as pltpu
from jax.experimental.pallas import tpu_sc as plsc

NUM_U = 5000
NUM_I = 5000
N = NUM_U + NUM_I
NP = 10240
D = 128
E = 320000
NC = 2
NS = 16
NW = NC * NS
CHUNK = 64
CH_PER_G = 32
G0 = 10
G1 = 0
NGT = NS * (G0 + G1)
E_PAD = NGT * CH_PER_G * CHUNK
ROWS_PER_TILE = NP // NS
NBUF = 4


def _sc_propagate(ego, cols, rows, vals, zeros):
    mesh = plsc.VectorSubcoreMesh(
        core_axis_name="c", subcore_axis_name="s",
        num_cores=NC, num_subcores=NS)

    @functools.partial(
        pl.kernel,
        out_type=jax.ShapeDtypeStruct((NC * NP, D), jnp.float32),
        mesh=mesh,
        compiler_params=pltpu.CompilerParams(use_tc_tiling_on_sc=False),
        scratch_types=[
            pltpu.VMEM((CH_PER_G, CHUNK), jnp.int32),
            pltpu.VMEM((2 * CH_PER_G, CHUNK // 2), jnp.int32),
            pltpu.VMEM((CH_PER_G, CHUNK), jnp.float32),
            [pltpu.VMEM((CHUNK, D // 2), jnp.int32) for _ in range(NBUF)],
            [pltpu.VMEM((CHUNK // 2, D), jnp.float32) for _ in range(NBUF)],
            pltpu.VMEM_SHARED((NP, D), jnp.float32),
            [pltpu.SemaphoreType.DMA for _ in range(NBUF)],
            [pltpu.SemaphoreType.DMA for _ in range(NBUF)],
        ],
    )
    def k(ego_h, cols_h, rows_h, vals_h, zeros_h, part_h,
          colg, rowg, valg, ibufs, obufs, acc, gsems, ssems):
        c = lax.axis_index("c")
        s = lax.axis_index("s")
        ng = jnp.where(c == 0, G0, G1)
        gbase = jnp.where(c == 0, s * G0, NS * G0 + s * G1)
        pltpu.sync_copy(zeros_h, acc.at[pl.ds(s * ROWS_PER_TILE, ROWS_PER_TILE)])
        plsc.subcore_barrier()

        def swait(b):
            pltpu.make_async_copy(obufs[b], acc.at[rowg.at[0]], ssems[b]).wait()

        def grp(g, carry):
            @pl.when(g > 0)
            def _drain_prev_tail():
                for b in range(NBUF):
                    swait(b)

            gi = gbase + g
            pltpu.sync_copy(cols_h.at[gi], colg)
            pltpu.sync_copy(rows_h.at[gi], rowg)
            pltpu.sync_copy(vals_h.at[gi], valg)
            for b in range(2):
                pltpu.async_copy(ego_h.at[colg.at[b]], ibufs[b], gsems[b])

            def quad(i4, carry2):
                for b in range(NBUF):
                    il = i4 * NBUF + b
                    bj = (b + 2) % NBUF

                    @pl.when(il + 2 < CH_PER_G)
                    def _issue_next_gather():
                        pltpu.async_copy(ego_h.at[colg.at[il + 2]],
                                         ibufs[bj], gsems[bj])

                    pltpu.make_async_copy(
                        ego_h.at[colg.at[il]], ibufs[b], gsems[b]).wait()

                    for h in range(2):
                        o = (2 * b + h) % NBUF

                        @pl.when(il >= 2)
                        def _wait_obuf_free():
                            pltpu.make_async_copy(
                                obufs[o], acc.at[rowg.at[0]], ssems[o]).wait()

                        def sixteen(gg, carry3):
                            vals16 = valg[il, pl.ds(h * 32 + gg * 16, 16)]
                            for kk in range(16):
                                v = vals16[kk]
                                e = h * 32 + gg * 16 + kk
                                eo = gg * 16 + kk
                                for j in range(D // 32):
                                    u = ibufs[b][e, pl.ds(j * 16, 16)]
                                    lo = lax.bitcast_convert_type(
                                        lax.shift_left(u, 16), jnp.float32)
                                    hi = lax.bitcast_convert_type(
                                        u & (-65536), jnp.float32)
                                    obufs[o][eo, pl.ds(j * 16, 16)] = lo * v
                                    obufs[o][eo, pl.ds(D // 2 + j * 16, 16)] = (
                                        hi * v)
                            return carry3

                        lax.fori_loop(0, CHUNK // 32, sixteen, 0)
                        pltpu.async_copy(
                            obufs[o], acc.at[rowg.at[2 * il + h]], ssems[o],
                            add=True)
                return carry2

            lax.fori_loop(0, CH_PER_G // NBUF, quad, 0)
            return carry

        lax.fori_loop(0, ng, grp, 0)

        @pl.when(ng > 0)
        def _drain_final_tail():
            for b in range(NBUF):
                swait(b)

        plsc.subcore_barrier()
        r0 = s * ROWS_PER_TILE
        pltpu.sync_copy(acc.at[pl.ds(r0, ROWS_PER_TILE)],
                        part_h.at[pl.ds(c * NP + r0, ROWS_PER_TILE)])

    return k(ego, cols, rows, vals, zeros)


def _pack_pairs_f32_to_i32(x):
    lo = jax.lax.bitcast_convert_type(
        x[:, :D // 2].astype(jnp.bfloat16), jnp.uint16).astype(jnp.uint32)
    hi = jax.lax.bitcast_convert_type(
        x[:, D // 2:].astype(jnp.bfloat16), jnp.uint16).astype(jnp.uint32)
    return ((hi << 16) | lo).astype(jnp.int32)


def _tc_merge(part):
    blk = 640

    def body(a_ref, b_ref, o_ref, p_ref):
        s = a_ref[...] + b_ref[...]
        o_ref[...] = s
        p_ref[...] = _pack_pairs_f32_to_i32(s)

    return pl.pallas_call(
        body,
        grid=(NP // blk,),
        in_specs=[
            pl.BlockSpec((blk, D), lambda i: (i, 0)),
            pl.BlockSpec((blk, D), lambda i: (i + NP // blk, 0)),
        ],
        out_specs=[
            pl.BlockSpec((blk, D), lambda i: (i, 0)),
            pl.BlockSpec((blk, D // 2), lambda i: (i, 0)),
        ],
        out_shape=[
            jax.ShapeDtypeStruct((NP, D), jnp.float32),
            jax.ShapeDtypeStruct((NP, D // 2), jnp.int32),
        ],
    )(part, part)


def _tc_user_mean(u0, e1, e2, e3):
    blk = 200

    def body(a_ref, b_ref, c_ref, d_ref, o_ref):
        o_ref[...] = (a_ref[...] + b_ref[...] + c_ref[...] + d_ref[...]) * 0.25

    return pl.pallas_call(
        body,
        grid=(NUM_U // blk,),
        in_specs=[pl.BlockSpec((blk, D), lambda i: (i, 0))] * 4,
        out_specs=pl.BlockSpec((blk, D), lambda i: (i, 0)),
        out_shape=jax.ShapeDtypeStruct((NUM_U, D), jnp.float32),
    )(u0, e1, e2, e3)


def kernel(adj_indices, adj_values, user_emb, item_emb):
    row = adj_indices[0]
    col = adj_indices[1]
    pad = E_PAD - E
    spread = (jnp.arange(pad, dtype=jnp.int32) * 7) % NP
    rows = jnp.concatenate([row, spread])
    cols = jnp.concatenate([col, spread])
    vals = jnp.concatenate([adj_values, jnp.zeros((pad,), jnp.float32)])
    rows = rows.reshape(NGT, 2 * CH_PER_G, CHUNK // 2)
    cols = cols.reshape(NGT, CH_PER_G, CHUNK)
    vals = vals.reshape(NGT, CH_PER_G, CHUNK)
    zeros = jnp.zeros((ROWS_PER_TILE, D), jnp.float32)

    ego0 = jnp.concatenate(
        [user_emb, item_emb, jnp.zeros((NP - N, D), jnp.float32)], axis=0)
    egos = [ego0]
    packed = _pack_pairs_f32_to_i32(ego0)
    for _ in range(3):
        part = _sc_propagate(packed, cols, rows, vals, zeros)
        ego_k, packed = _tc_merge(part)
        egos.append(ego_k)

    user_out = _tc_user_mean(user_emb, egos[1], egos[2], egos[3])
    item_embs = (item_emb, egos[1][NUM_U:N], egos[2][NUM_U:N], egos[3][NUM_U:N])
    return (user_out, item_embs)


# speedup vs baseline: 4.3456x; 4.3456x over previous
import functools

import jax
import jax.numpy as jnp
from jax import lax
from jax.experimental import pallas as pl
from jax.experimental.pallas import tpu as pltpu
from jax.experimental.pallas import tpu_sc as plsc

NUM_U = 5000
NUM_I = 5000
N = NUM_U + NUM_I
NP = 10240
D = 128
E = 320000
NC = 2
NS = 16
NW = NC * NS
CHUNK = 64
CH_PER_G = 32
G0 = 5
G1 = 5
NGT = NS * (G0 + G1)
E_PAD = NGT * CH_PER_G * CHUNK
ROWS_PER_TILE = NP // NS
NBUF = 4


def _sc_propagate(ego, cols, rows, vals, zeros):
    mesh = plsc.VectorSubcoreMesh(
        core_axis_name="c", subcore_axis_name="s",
        num_cores=NC, num_subcores=NS)

    @functools.partial(
        pl.kernel,
        out_type=jax.ShapeDtypeStruct((NC * NP, D), jnp.float32),
        mesh=mesh,
        scratch_types=[
            pltpu.VMEM((CH_PER_G, CHUNK), jnp.int32),
            pltpu.VMEM((CH_PER_G, CHUNK), jnp.int32),
            pltpu.VMEM((CH_PER_G, CHUNK), jnp.float32),
            [pltpu.VMEM((CHUNK, D), jnp.float32) for _ in range(NBUF)],
            pltpu.VMEM_SHARED((NP, D), jnp.float32),
            [pltpu.SemaphoreType.DMA for _ in range(NBUF)],
            [pltpu.SemaphoreType.DMA for _ in range(NBUF)],
        ],
    )
    def k(ego_h, cols_h, rows_h, vals_h, zeros_h, part_h,
          colg, rowg, valg, bufs, acc, gsems, ssems):
        c = lax.axis_index("c")
        s = lax.axis_index("s")
        ng = jnp.where(c == 0, G0, G1)
        gbase = jnp.where(c == 0, s * G0, NS * G0 + s * G1)
        pltpu.sync_copy(zeros_h, acc.at[pl.ds(s * ROWS_PER_TILE, ROWS_PER_TILE)])
        plsc.subcore_barrier()

        def tail_wait(b):
            pltpu.make_async_copy(bufs[b], acc.at[rowg.at[0]], ssems[b]).wait()

        def grp(g, carry):
            @pl.when(g > 0)
            def _drain_prev_tail():
                tail_wait((CH_PER_G - 2) % NBUF)
                tail_wait((CH_PER_G - 1) % NBUF)

            gi = gbase + g
            pltpu.sync_copy(cols_h.at[gi], colg)
            pltpu.sync_copy(rows_h.at[gi], rowg)
            pltpu.sync_copy(vals_h.at[gi], valg)
            for b in range(2):
                pltpu.async_copy(ego_h.at[colg.at[b]], bufs[b], gsems[b])

            def quad(i4, carry2):
                for b in range(NBUF):
                    il = i4 * NBUF + b
                    bj = (b + 2) % NBUF

                    @pl.when(il >= 2)
                    def _wait_prev_scatter():
                        pltpu.make_async_copy(
                            bufs[bj], acc.at[rowg.at[il - 2]], ssems[bj]).wait()

                    @pl.when(il + 2 < CH_PER_G)
                    def _issue_next_gather():
                        pltpu.async_copy(ego_h.at[colg.at[il + 2]],
                                         bufs[bj], gsems[bj])

                    pltpu.make_async_copy(
                        ego_h.at[colg.at[il]], bufs[b], gsems[b]).wait()

                    def sixteen(gg, carry3):
                        vals16 = valg[il, pl.ds(gg * 16, 16)]
                        for kk in range(16):
                            v = vals16[kk]
                            e = gg * 16 + kk
                            for j in range(D // 16):
                                sl = pl.ds(j * 16, 16)
                                bufs[b][e, sl] = bufs[b][e, sl] * v
                        return carry3

                    lax.fori_loop(0, CHUNK // 16, sixteen, 0)
                    pltpu.async_copy(bufs[b], acc.at[rowg.at[il]], ssems[b],
                                     add=True)
                return carry2

            lax.fori_loop(0, CH_PER_G // NBUF, quad, 0)
            return carry

        lax.fori_loop(0, ng, grp, 0)

        @pl.when(ng > 0)
        def _drain_final_tail():
            tail_wait((CH_PER_G - 2) % NBUF)
            tail_wait((CH_PER_G - 1) % NBUF)
        plsc.subcore_barrier()
        r0 = s * ROWS_PER_TILE
        pltpu.sync_copy(acc.at[pl.ds(r0, ROWS_PER_TILE)],
                        part_h.at[pl.ds(c * NP + r0, ROWS_PER_TILE)])

    return k(ego, cols, rows, vals, zeros)


def _tc_merge(part):
    blk = 640

    def body(a_ref, b_ref, o_ref):
        o_ref[...] = a_ref[...] + b_ref[...]

    return pl.pallas_call(
        body,
        grid=(NP // blk,),
        in_specs=[
            pl.BlockSpec((blk, D), lambda i: (i, 0)),
            pl.BlockSpec((blk, D), lambda i: (i + NP // blk, 0)),
        ],
        out_specs=pl.BlockSpec((blk, D), lambda i: (i, 0)),
        out_shape=jax.ShapeDtypeStruct((NP, D), jnp.float32),
    )(part, part)


def _tc_user_mean(u0, e1, e2, e3):
    blk = 200

    def body(a_ref, b_ref, c_ref, d_ref, o_ref):
        o_ref[...] = (a_ref[...] + b_ref[...] + c_ref[...] + d_ref[...]) * 0.25

    return pl.pallas_call(
        body,
        grid=(NUM_U // blk,),
        in_specs=[pl.BlockSpec((blk, D), lambda i: (i, 0))] * 4,
        out_specs=pl.BlockSpec((blk, D), lambda i: (i, 0)),
        out_shape=jax.ShapeDtypeStruct((NUM_U, D), jnp.float32),
    )(u0, e1, e2, e3)


def kernel(adj_indices, adj_values, user_emb, item_emb):
    row = adj_indices[0]
    col = adj_indices[1]
    pad = E_PAD - E
    spread = (jnp.arange(pad, dtype=jnp.int32) * 7) % NP
    rows = jnp.concatenate([row, spread])
    cols = jnp.concatenate([col, spread])
    vals = jnp.concatenate([adj_values, jnp.zeros((pad,), jnp.float32)])
    rows = rows.reshape(NGT, CH_PER_G, CHUNK)
    cols = cols.reshape(NGT, CH_PER_G, CHUNK)
    vals = vals.reshape(NGT, CH_PER_G, CHUNK)
    zeros = jnp.zeros((ROWS_PER_TILE, D), jnp.float32)

    ego0 = jnp.concatenate(
        [user_emb, item_emb, jnp.zeros((NP - N, D), jnp.float32)], axis=0)
    egos = [ego0]
    for _ in range(3):
        part = _sc_propagate(egos[-1], cols, rows, vals, zeros)
        egos.append(_tc_merge(part))

    user_out = _tc_user_mean(user_emb, egos[1], egos[2], egos[3])
    item_embs = (item_emb, egos[1][NUM_U:N], egos[2][NUM_U:N], egos[3][NUM_U:N])
    return (user_out, item_embs)


# speedup vs baseline: 4.4197x; 1.0171x over previous
import functools

import jax
import jax.numpy as jnp
from jax import lax
from jax.experimental import pallas as pl
from jax.experimental.pallas import tpu as pltpu
from jax.experimental.pallas import tpu_sc as plsc

NUM_U = 5000
NUM_I = 5000
N = NUM_U + NUM_I
NP = 10240
D = 128
E = 320000
NC = 2
NS = 16
NW = NC * NS
CHUNK = 64
CH_PER_G = 40
G0 = 4
G1 = 4
NGT = NS * (G0 + G1)
E_PAD = NGT * CH_PER_G * CHUNK
ROWS_PER_TILE = NP // NS
NBUF = 4


def _sc_propagate(ego, cols, rows, vals, zeros):
    mesh = plsc.VectorSubcoreMesh(
        core_axis_name="c", subcore_axis_name="s",
        num_cores=NC, num_subcores=NS)

    @functools.partial(
        pl.kernel,
        out_type=jax.ShapeDtypeStruct((NC * NP, D), jnp.float32),
        mesh=mesh,
        scratch_types=[
            pltpu.VMEM((CH_PER_G, CHUNK), jnp.int32),
            pltpu.VMEM((CH_PER_G, CHUNK), jnp.int32),
            pltpu.VMEM((CH_PER_G, CHUNK), jnp.float32),
            [pltpu.VMEM((CHUNK, D), jnp.float32) for _ in range(NBUF)],
            pltpu.VMEM_SHARED((NP, D), jnp.float32),
            [pltpu.SemaphoreType.DMA for _ in range(NBUF)],
            [pltpu.SemaphoreType.DMA for _ in range(NBUF)],
        ],
    )
    def k(ego_h, cols_h, rows_h, vals_h, zeros_h, part_h,
          colg, rowg, valg, bufs, acc, gsems, ssems):
        c = lax.axis_index("c")
        s = lax.axis_index("s")
        ng = jnp.where(c == 0, G0, G1)
        gbase = jnp.where(c == 0, s * G0, NS * G0 + s * G1)
        pltpu.sync_copy(zeros_h, acc.at[pl.ds(s * ROWS_PER_TILE, ROWS_PER_TILE)])
        plsc.subcore_barrier()

        def tail_wait(b):
            pltpu.make_async_copy(bufs[b], acc.at[rowg.at[0]], ssems[b]).wait()

        def grp(g, carry):
            @pl.when(g > 0)
            def _drain_prev_tail():
                tail_wait((CH_PER_G - 2) % NBUF)
                tail_wait((CH_PER_G - 1) % NBUF)

            gi = gbase + g
            pltpu.sync_copy(cols_h.at[gi], colg)
            pltpu.sync_copy(rows_h.at[gi], rowg)
            pltpu.sync_copy(vals_h.at[gi], valg)
            for b in range(2):
                pltpu.async_copy(ego_h.at[colg.at[b]], bufs[b], gsems[b])

            def quad(i4, carry2):
                for b in range(NBUF):
                    il = i4 * NBUF + b
                    bj = (b + 2) % NBUF

                    @pl.when(il >= 2)
                    def _wait_prev_scatter():
                        pltpu.make_async_copy(
                            bufs[bj], acc.at[rowg.at[il - 2]], ssems[bj]).wait()

                    @pl.when(il + 2 < CH_PER_G)
                    def _issue_next_gather():
                        pltpu.async_copy(ego_h.at[colg.at[il + 2]],
                                         bufs[bj], gsems[bj])

                    pltpu.make_async_copy(
                        ego_h.at[colg.at[il]], bufs[b], gsems[b]).wait()

                    def sixteen(gg, carry3):
                        vals16 = valg[il, pl.ds(gg * 16, 16)]
                        for kk in range(16):
                            v = vals16[kk]
                            e = gg * 16 + kk
                            for j in range(D // 16):
                                sl = pl.ds(j * 16, 16)
                                bufs[b][e, sl] = bufs[b][e, sl] * v
                        return carry3

                    lax.fori_loop(0, CHUNK // 16, sixteen, 0)
                    pltpu.async_copy(bufs[b], acc.at[rowg.at[il]], ssems[b],
                                     add=True)
                return carry2

            lax.fori_loop(0, CH_PER_G // NBUF, quad, 0)
            return carry

        lax.fori_loop(0, ng, grp, 0)

        @pl.when(ng > 0)
        def _drain_final_tail():
            tail_wait((CH_PER_G - 2) % NBUF)
            tail_wait((CH_PER_G - 1) % NBUF)
        plsc.subcore_barrier()
        r0 = s * ROWS_PER_TILE
        pltpu.sync_copy(acc.at[pl.ds(r0, ROWS_PER_TILE)],
                        part_h.at[pl.ds(c * NP + r0, ROWS_PER_TILE)])

    return k(ego, cols, rows, vals, zeros)


def _tc_merge(part):
    blk = 640

    def body(a_ref, b_ref, o_ref):
        o_ref[...] = a_ref[...] + b_ref[...]

    return pl.pallas_call(
        body,
        grid=(NP // blk,),
        in_specs=[
            pl.BlockSpec((blk, D), lambda i: (i, 0)),
            pl.BlockSpec((blk, D), lambda i: (i + NP // blk, 0)),
        ],
        out_specs=pl.BlockSpec((blk, D), lambda i: (i, 0)),
        out_shape=jax.ShapeDtypeStruct((NP, D), jnp.float32),
    )(part, part)


def _tc_user_mean(u0, e1, e2, e3):
    blk = 200

    def body(a_ref, b_ref, c_ref, d_ref, o_ref):
        o_ref[...] = (a_ref[...] + b_ref[...] + c_ref[...] + d_ref[...]) * 0.25

    return pl.pallas_call(
        body,
        grid=(NUM_U // blk,),
        in_specs=[pl.BlockSpec((blk, D), lambda i: (i, 0))] * 4,
        out_specs=pl.BlockSpec((blk, D), lambda i: (i, 0)),
        out_shape=jax.ShapeDtypeStruct((NUM_U, D), jnp.float32),
    )(u0, e1, e2, e3)


def kernel(adj_indices, adj_values, user_emb, item_emb):
    row = adj_indices[0]
    col = adj_indices[1]
    pad = E_PAD - E
    spread = (jnp.arange(pad, dtype=jnp.int32) * 7) % NP
    rows = jnp.concatenate([row, spread])
    cols = jnp.concatenate([col, spread])
    vals = jnp.concatenate([adj_values, jnp.zeros((pad,), jnp.float32)])
    rows = rows.reshape(NGT, CH_PER_G, CHUNK)
    cols = cols.reshape(NGT, CH_PER_G, CHUNK)
    vals = vals.reshape(NGT, CH_PER_G, CHUNK)
    zeros = jnp.zeros((ROWS_PER_TILE, D), jnp.float32)

    ego0 = jnp.concatenate(
        [user_emb, item_emb, jnp.zeros((NP - N, D), jnp.float32)], axis=0)
    egos = [ego0]
    for _ in range(3):
        part = _sc_propagate(egos[-1], cols, rows, vals, zeros)
        egos.append(_tc_merge(part))

    user_out = _tc_user_mean(user_emb, egos[1], egos[2], egos[3])
    item_embs = (item_emb, egos[1][NUM_U:N], egos[2][NUM_U:N], egos[3][NUM_U:N])
    return (user_out, item_embs)


# speedup vs baseline: 4.4208x; 1.0002x over previous
import functools

import jax
import jax.numpy as jnp
from jax import lax
from jax.experimental import pallas as pl
from jax.experimental.pallas import tpu as pltpu
from jax.experimental.pallas import tpu_sc as plsc

NUM_U = 5000
NUM_I = 5000
N = NUM_U + NUM_I
NP = 10240
D = 128
E = 320000
NC = 2
NS = 16
NW = NC * NS
CHUNK = 64
CH_PER_G = 40
G0 = 4
G1 = 4
NGT = NS * (G0 + G1)
E_PAD = NGT * CH_PER_G * CHUNK
ROWS_PER_TILE = NP // NS
NBUF = 4


def _sc_propagate(ego, cols, rows, vals, zeros):
    mesh = plsc.VectorSubcoreMesh(
        core_axis_name="c", subcore_axis_name="s",
        num_cores=NC, num_subcores=NS)

    @functools.partial(
        pl.kernel,
        out_type=jax.ShapeDtypeStruct((NC * NP, D), jnp.float32),
        mesh=mesh,
        scratch_types=[
            pltpu.VMEM((CH_PER_G, CHUNK), jnp.int32),
            pltpu.VMEM((CH_PER_G, CHUNK), jnp.int32),
            pltpu.VMEM((CH_PER_G, CHUNK), jnp.float32),
            [pltpu.VMEM((CHUNK, D), jnp.float32) for _ in range(NBUF)],
            pltpu.VMEM_SHARED((NP, D), jnp.float32),
            [pltpu.SemaphoreType.DMA for _ in range(NBUF)],
            [pltpu.SemaphoreType.DMA for _ in range(NBUF)],
        ],
    )
    def k(ego_h, cols_h, rows_h, vals_h, zeros_h, part_h,
          colg, rowg, valg, bufs, acc, gsems, ssems):
        c = lax.axis_index("c")
        s = lax.axis_index("s")
        ng = jnp.where(c == 0, G0, G1)
        gbase = jnp.where(c == 0, s * G0, NS * G0 + s * G1)
        pltpu.sync_copy(zeros_h, acc.at[pl.ds(s * ROWS_PER_TILE, ROWS_PER_TILE)])
        plsc.subcore_barrier()

        def tail_wait(b):
            pltpu.make_async_copy(bufs[b], acc.at[rowg.at[0]], ssems[b]).wait()

        def grp(g, carry):
            @pl.when(g > 0)
            def _drain_prev_tail():
                tail_wait((CH_PER_G - 2) % NBUF)
                tail_wait((CH_PER_G - 1) % NBUF)

            gi = gbase + g
            pltpu.sync_copy(cols_h.at[gi], colg)
            pltpu.sync_copy(rows_h.at[gi], rowg)
            pltpu.sync_copy(vals_h.at[gi], valg)
            for b in range(2):
                pltpu.async_copy(ego_h.at[colg.at[b]], bufs[b], gsems[b])

            def quad(i4, carry2):
                for b in range(NBUF):
                    il = i4 * NBUF + b
                    bj = (b + 2) % NBUF

                    @pl.when(il >= 2)
                    def _wait_prev_scatter():
                        pltpu.make_async_copy(
                            bufs[bj], acc.at[rowg.at[il - 2]], ssems[bj]).wait()

                    @pl.when(il + 2 < CH_PER_G)
                    def _issue_next_gather():
                        pltpu.async_copy(ego_h.at[colg.at[il + 2]],
                                         bufs[bj], gsems[bj])

                    pltpu.make_async_copy(
                        ego_h.at[colg.at[il]], bufs[b], gsems[b]).wait()

                    def sixteen(gg, carry3):
                        vals16 = valg[il, pl.ds(gg * 16, 16)]
                        for kk in range(16):
                            v = vals16[kk]
                            e = gg * 16 + kk
                            for j in range(D // 16):
                                sl = pl.ds(j * 16, 16)
                                bufs[b][e, sl] = bufs[b][e, sl] * v
                        return carry3

                    lax.fori_loop(0, CHUNK // 16, sixteen, 0)
                    pltpu.async_copy(bufs[b], acc.at[rowg.at[il]], ssems[b],
                                     add=True)
                return carry2

            lax.fori_loop(0, CH_PER_G // NBUF, quad, 0)
            return carry

        lax.fori_loop(0, ng, grp, 0)

        @pl.when(ng > 0)
        def _drain_final_tail():
            tail_wait((CH_PER_G - 2) % NBUF)
            tail_wait((CH_PER_G - 1) % NBUF)
        plsc.subcore_barrier()
        r0 = s * ROWS_PER_TILE
        pltpu.sync_copy(acc.at[pl.ds(r0, ROWS_PER_TILE)],
                        part_h.at[pl.ds(c * NP + r0, ROWS_PER_TILE)])

    return k(ego, cols, rows, vals, zeros)


def _tc_merge(part):
    blk = 640

    def body(a_ref, b_ref, o_ref):
        o_ref[...] = a_ref[...] + b_ref[...]

    return pl.pallas_call(
        body,
        grid=(NP // blk,),
        in_specs=[
            pl.BlockSpec((blk, D), lambda i: (i, 0)),
            pl.BlockSpec((blk, D), lambda i: (i + NP // blk, 0)),
        ],
        out_specs=pl.BlockSpec((blk, D), lambda i: (i, 0)),
        out_shape=jax.ShapeDtypeStruct((NP, D), jnp.float32),
    )(part, part)


def _tc_user_mean(u0, e1, e2, e3):
    blk = 200

    def body(a_ref, b_ref, c_ref, d_ref, o_ref):
        o_ref[...] = (a_ref[...] + b_ref[...] + c_ref[...] + d_ref[...]) * 0.25

    return pl.pallas_call(
        body,
        grid=(NUM_U // blk,),
        in_specs=[pl.BlockSpec((blk, D), lambda i: (i, 0))] * 4,
        out_specs=pl.BlockSpec((blk, D), lambda i: (i, 0)),
        out_shape=jax.ShapeDtypeStruct((NUM_U, D), jnp.float32),
    )(u0, e1, e2, e3)


def kernel(adj_indices, adj_values, user_emb, item_emb):
    row = adj_indices[0]
    col = adj_indices[1]
    pad = E_PAD - E
    spread = jnp.arange(pad, dtype=jnp.int32)
    rows = jnp.concatenate([row, spread])
    cols = jnp.concatenate([col, spread])
    vals = jnp.concatenate([adj_values, jnp.zeros((pad,), jnp.float32)])
    rows = rows.reshape(NGT, CH_PER_G, CHUNK)
    cols = cols.reshape(NGT, CH_PER_G, CHUNK)
    vals = vals.reshape(NGT, CH_PER_G, CHUNK)
    zeros = jnp.zeros((ROWS_PER_TILE, D), jnp.float32)

    ego0 = jnp.concatenate(
        [user_emb, item_emb, jnp.zeros((NP - N, D), jnp.float32)], axis=0)
    egos = [ego0]
    for _ in range(3):
        part = _sc_propagate(egos[-1], cols, rows, vals, zeros)
        egos.append(_tc_merge(part))

    user_out = _tc_user_mean(user_emb, egos[1], egos[2], egos[3])
    item_embs = (item_emb, egos[1][NUM_U:N], egos[2][NUM_U:N], egos[3][NUM_U:N])
    return (user_out, item_embs)


# speedup vs baseline: 4.4230x; 1.0005x over previous
import functools

import jax
import jax.numpy as jnp
from jax import lax
from jax.experimental import pallas as pl
from jax.experimental.pallas import tpu as pltpu
from jax.experimental.pallas import tpu_sc as plsc

NUM_U = 5000
NUM_I = 5000
N = NUM_U + NUM_I
NP = 10240
D = 128
E = 320000
NC = 2
NS = 16
CHUNK = 64
CH_PER_G = 40
G0 = 4
G1 = 4
NGT = NS * (G0 + G1)
E_PAD = NGT * CH_PER_G * CHUNK
ROWS_PER_TILE = NP // NS
NBUF = 4


def _sc_propagate(ego, cols, rows, vals, zeros):
    mesh = plsc.VectorSubcoreMesh(
        core_axis_name="c", subcore_axis_name="s",
        num_cores=NC, num_subcores=NS)

    @functools.partial(
        pl.kernel,
        out_type=jax.ShapeDtypeStruct((NC * NP, D), jnp.float32),
        mesh=mesh,
        scratch_types=[
            pltpu.VMEM((CH_PER_G, CHUNK), jnp.int32),
            pltpu.VMEM((CH_PER_G, CHUNK), jnp.int32),
            pltpu.VMEM((CH_PER_G, CHUNK), jnp.float32),
            [pltpu.VMEM((CHUNK, D), jnp.float32) for _ in range(NBUF)],
            pltpu.VMEM_SHARED((NP, D), jnp.float32),
            [pltpu.SemaphoreType.DMA for _ in range(NBUF)],
            [pltpu.SemaphoreType.DMA for _ in range(NBUF)],
        ],
    )
    def k(ego_h, cols_h, rows_h, vals_h, zeros_h, part_h,
          colg, rowg, valg, bufs, acc, gsems, ssems):
        c = lax.axis_index("c")
        s = lax.axis_index("s")
        ng = jnp.where(c == 0, G0, G1)
        gbase = jnp.where(c == 0, s * G0, NS * G0 + s * G1)
        pltpu.sync_copy(zeros_h, acc.at[pl.ds(s * ROWS_PER_TILE, ROWS_PER_TILE)])
        plsc.subcore_barrier()

        def tail_wait(b):
            pltpu.make_async_copy(bufs[b], acc.at[rowg.at[0]], ssems[b]).wait()

        def grp(g, carry):
            @pl.when(g > 0)
            def _drain_prev_tail():
                tail_wait((CH_PER_G - 2) % NBUF)
                tail_wait((CH_PER_G - 1) % NBUF)

            gi = gbase + g
            pltpu.sync_copy(cols_h.at[gi], colg)
            pltpu.sync_copy(rows_h.at[gi], rowg)
            pltpu.sync_copy(vals_h.at[gi], valg)
            for b in range(2):
                pltpu.async_copy(ego_h.at[colg.at[b]], bufs[b], gsems[b])

            def quad(i4, carry2):
                for b in range(NBUF):
                    il = i4 * NBUF + b
                    bj = (b + 2) % NBUF

                    @pl.when(il >= 2)
                    def _wait_prev_scatter():
                        pltpu.make_async_copy(
                            bufs[bj], acc.at[rowg.at[il - 2]], ssems[bj]).wait()

                    @pl.when(il + 2 < CH_PER_G)
                    def _issue_next_gather():
                        pltpu.async_copy(ego_h.at[colg.at[il + 2]],
                                         bufs[bj], gsems[bj])

                    pltpu.make_async_copy(
                        ego_h.at[colg.at[il]], bufs[b], gsems[b]).wait()

                    def sixteen(gg, carry3):
                        vals16 = valg[il, pl.ds(gg * 16, 16)]
                        for kk in range(16):
                            v = vals16[kk]
                            e = gg * 16 + kk
                            for j in range(D // 16):
                                sl = pl.ds(j * 16, 16)
                                bufs[b][e, sl] = bufs[b][e, sl] * v
                        return carry3

                    lax.fori_loop(0, CHUNK // 16, sixteen, 0)
                    pltpu.async_copy(bufs[b], acc.at[rowg.at[il]], ssems[b],
                                     add=True)
                return carry2

            lax.fori_loop(0, CH_PER_G // NBUF, quad, 0)
            return carry

        lax.fori_loop(0, ng, grp, 0)

        @pl.when(ng > 0)
        def _drain_final_tail():
            tail_wait((CH_PER_G - 2) % NBUF)
            tail_wait((CH_PER_G - 1) % NBUF)
        plsc.subcore_barrier()
        r0 = s * ROWS_PER_TILE
        pltpu.sync_copy(acc.at[pl.ds(r0, ROWS_PER_TILE)],
                        part_h.at[pl.ds(c * NP + r0, ROWS_PER_TILE)])

    return k(ego, cols, rows, vals, zeros)


def _tc_merge(part):
    blk = 640

    def body(a_ref, b_ref, o_ref):
        o_ref[...] = a_ref[...] + b_ref[...]

    return pl.pallas_call(
        body,
        grid=(NP // blk,),
        in_specs=[
            pl.BlockSpec((blk, D), lambda i: (i, 0)),
            pl.BlockSpec((blk, D), lambda i: (i + NP // blk, 0)),
        ],
        out_specs=pl.BlockSpec((blk, D), lambda i: (i, 0)),
        out_shape=jax.ShapeDtypeStruct((NP, D), jnp.float32),
    )(part, part)


def _tc_user_mean(u0, e1, e2, e3):
    blk = 200

    def body(a_ref, b_ref, c_ref, d_ref, o_ref):
        o_ref[...] = (a_ref[...] + b_ref[...] + c_ref[...] + d_ref[...]) * 0.25

    return pl.pallas_call(
        body,
        grid=(NUM_U // blk,),
        in_specs=[pl.BlockSpec((blk, D), lambda i: (i, 0))] * 4,
        out_specs=pl.BlockSpec((blk, D), lambda i: (i, 0)),
        out_shape=jax.ShapeDtypeStruct((NUM_U, D), jnp.float32),
    )(u0, e1, e2, e3)


def kernel(adj_indices, adj_values, user_emb, item_emb):
    row = adj_indices[0]
    col = adj_indices[1]
    pad = E_PAD - E
    spread = jnp.arange(pad, dtype=jnp.int32)
    rows = jnp.concatenate([row, spread])
    cols = jnp.concatenate([col, spread])
    vals = jnp.concatenate([adj_values, jnp.zeros((pad,), jnp.float32)])
    rows = rows.reshape(NGT, CH_PER_G, CHUNK)
    cols = cols.reshape(NGT, CH_PER_G, CHUNK)
    vals = vals.reshape(NGT, CH_PER_G, CHUNK)
    zeros = jnp.zeros((ROWS_PER_TILE, D), jnp.float32)

    ego0 = jnp.concatenate(
        [user_emb, item_emb, jnp.zeros((NP - N, D), jnp.float32)], axis=0)
    egos = [ego0]
    for _ in range(3):
        part = _sc_propagate(egos[-1], cols, rows, vals, zeros)
        egos.append(_tc_merge(part))

    user_out = _tc_user_mean(user_emb, egos[1], egos[2], egos[3])
    item_embs = (item_emb, egos[1][NUM_U:N], egos[2][NUM_U:N], egos[3][NUM_U:N])
    return (user_out, item_embs)
